# baseline (speedup 1.0000x reference)
import jax
import jax.numpy as jnp
from jax import lax
from jax.experimental import pallas as pl

_EPS = 1e-10
_ROWS = 8
_V = 100000
_CW = 512
_NFULL = _V // _CW
_TAIL = _V - _NFULL * _CW
_BIG = 2**31 - 1


def _fused_body(logits_ref, noise_ref, samples_ref, sel_ref):
    r = _ROWS
    neg_inf = jnp.float32(-jnp.inf)

    def gumbel_perturb(x, n):
        t = jnp.log(n + _EPS)
        w = jnp.log(_EPS - t)
        return x - w

    def loop_a(i, carry):
        bp, bi, bx, m = carry
        col0 = pl.multiple_of(i * _CW, _CW)
        x = logits_ref[:, pl.ds(col0, _CW)]
        n = noise_ref[:, pl.ds(col0, _CW)]
        p = gumbel_perturb(x, n)
        idx = lax.broadcasted_iota(jnp.int32, (r, _CW), 1) + col0
        upd = p > bp
        bp = jnp.where(upd, p, bp)
        bi = jnp.where(upd, idx, bi)
        bx = jnp.where(upd, x, bx)
        m = jnp.maximum(m, x)
        return bp, bi, bx, m

    init = (
        jnp.full((r, _CW), neg_inf, jnp.float32),
        jnp.zeros((r, _CW), jnp.int32),
        jnp.zeros((r, _CW), jnp.float32),
        jnp.full((r, _CW), neg_inf, jnp.float32),
    )
    bp, bi, bx, m = lax.fori_loop(0, _NFULL, loop_a, init)

    x_t = logits_ref[:, pl.ds(_NFULL * _CW, _TAIL)]
    n_t = noise_ref[:, pl.ds(_NFULL * _CW, _TAIL)]
    p_t = gumbel_perturb(x_t, n_t)
    iota_t = lax.broadcasted_iota(jnp.int32, (r, _TAIL), 1) + _NFULL * _CW
    pmax_t = jnp.max(p_t, axis=-1, keepdims=True)
    hit_t = p_t == pmax_t
    idx_t = jnp.min(jnp.where(hit_t, iota_t, _BIG), axis=-1, keepdims=True)
    sel_t = jnp.max(jnp.where(iota_t == idx_t, x_t, neg_inf), axis=-1,
                    keepdims=True)
    m_t = jnp.max(x_t, axis=-1, keepdims=True)

    pmax = jnp.max(bp, axis=-1, keepdims=True)
    hit = bp == pmax
    idx_main = jnp.min(jnp.where(hit, bi, _BIG), axis=-1, keepdims=True)
    sel_main = jnp.max(
        jnp.where(hit & (bi == idx_main), bx, neg_inf), axis=-1,
        keepdims=True)

    main_wins = pmax >= pmax_t
    idx_row = jnp.where(main_wins, idx_main, idx_t)
    sel_logit = jnp.where(main_wins, sel_main, sel_t)
    m_row = jnp.maximum(jnp.max(m, axis=-1, keepdims=True), m_t)

    def loop_b(i, s):
        col0 = pl.multiple_of(i * _CW, _CW)
        x = logits_ref[:, pl.ds(col0, _CW)]
        return s + jnp.exp(x - m_row)

    s = lax.fori_loop(0, _NFULL, loop_b,
                      jnp.zeros((r, _CW), jnp.float32))
    s_row = (jnp.sum(s, axis=-1, keepdims=True)
             + jnp.sum(jnp.exp(x_t - m_row), axis=-1, keepdims=True))

    samples_ref[...] = idx_row
    sel_ref[...] = sel_logit - m_row - jnp.log(s_row)


def kernel(logits, noise):
    b, v = logits.shape
    samples2, sel2 = pl.pallas_call(
        _fused_body,
        grid=(b // _ROWS,),
        in_specs=[
            pl.BlockSpec((_ROWS, v), lambda i: (i, 0)),
            pl.BlockSpec((_ROWS, v), lambda i: (i, 0)),
        ],
        out_specs=[
            pl.BlockSpec((_ROWS, 1), lambda i: (i, 0)),
            pl.BlockSpec((_ROWS, 1), lambda i: (i, 0)),
        ],
        out_shape=[
            jax.ShapeDtypeStruct((b, 1), jnp.int32),
            jax.ShapeDtypeStruct((b, 1), jnp.float32),
        ],
    )(logits, noise)
    return samples2[:, 0], sel2[:, 0]


# speedup vs baseline: 2.0827x; 2.0827x over previous
import jax
import jax.numpy as jnp
from jax import lax
from jax.experimental import pallas as pl

_EPS = 1e-10
_ROWS = 8
_V = 100000
_CW = 512
_UNROLL = 4
_OW = _CW * _UNROLL
_NOUT = _V // _OW
_NEXTRA = (_V - _NOUT * _OW) // _CW
_TAIL = _V - _NOUT * _OW - _NEXTRA * _CW
_BIG = 2**31 - 1


def _fused_body(logits_ref, noise_ref, samples_ref, sel_ref):
    r = _ROWS
    neg_inf = jnp.float32(-jnp.inf)

    def gumbel_perturb(x, n):
        t = jnp.log(n + _EPS)
        w = jnp.log(_EPS - t)
        return x - w

    def update(c, carry):
        bp, bc, bx, m = carry
        col0 = pl.multiple_of(c * _CW, _CW)
        x = logits_ref[:, pl.ds(col0, _CW)]
        n = noise_ref[:, pl.ds(col0, _CW)]
        p = gumbel_perturb(x, n)
        upd = p > bp
        bp = jnp.where(upd, p, bp)
        bc = jnp.where(upd, c, bc)
        bx = jnp.where(upd, x, bx)
        m = jnp.maximum(m, x)
        return bp, bc, bx, m

    def loop_a(i, carry):
        for j in range(_UNROLL):
            carry = update(i * _UNROLL + j, carry)
        return carry

    init = (
        jnp.full((r, _CW), neg_inf, jnp.float32),
        jnp.zeros((r, _CW), jnp.int32),
        jnp.zeros((r, _CW), jnp.float32),
        jnp.full((r, _CW), neg_inf, jnp.float32),
    )
    carry = lax.fori_loop(0, _NOUT, loop_a, init)
    for j in range(_NEXTRA):
        carry = update(_NOUT * _UNROLL + j, carry)
    bp, bc, bx, m = carry

    tail0 = _V - _TAIL
    x_t = logits_ref[:, pl.ds(tail0, _TAIL)]
    n_t = noise_ref[:, pl.ds(tail0, _TAIL)]
    p_t = gumbel_perturb(x_t, n_t)
    iota_t = lax.broadcasted_iota(jnp.int32, (r, _TAIL), 1) + tail0
    pmax_t = jnp.max(p_t, axis=-1, keepdims=True)
    hit_t = p_t == pmax_t
    idx_t = jnp.min(jnp.where(hit_t, iota_t, _BIG), axis=-1, keepdims=True)
    sel_t = jnp.max(jnp.where(iota_t == idx_t, x_t, neg_inf), axis=-1,
                    keepdims=True)
    m_t = jnp.max(x_t, axis=-1, keepdims=True)

    bi = bc * _CW + lax.broadcasted_iota(jnp.int32, (r, _CW), 1)
    pmax = jnp.max(bp, axis=-1, keepdims=True)
    hit = bp == pmax
    idx_main = jnp.min(jnp.where(hit, bi, _BIG), axis=-1, keepdims=True)
    sel_main = jnp.max(
        jnp.where(hit & (bi == idx_main), bx, neg_inf), axis=-1,
        keepdims=True)

    main_wins = pmax >= pmax_t
    idx_row = jnp.where(main_wins, idx_main, idx_t)
    sel_logit = jnp.where(main_wins, sel_main, sel_t)
    m_row = jnp.maximum(jnp.max(m, axis=-1, keepdims=True), m_t)

    def loop_b(i, s):
        col0 = pl.multiple_of(i * _OW, _OW)
        x = logits_ref[:, pl.ds(col0, _OW)]
        return s + jnp.exp(x - m_row)

    s = lax.fori_loop(0, _NOUT, loop_b,
                      jnp.zeros((r, _OW), jnp.float32))
    s_row = jnp.sum(s, axis=-1, keepdims=True)
    for j in range(_NEXTRA):
        col0 = (_NOUT * _UNROLL + j) * _CW
        x = logits_ref[:, pl.ds(col0, _CW)]
        s_row = s_row + jnp.sum(jnp.exp(x - m_row), axis=-1, keepdims=True)
    s_row = s_row + jnp.sum(jnp.exp(x_t - m_row), axis=-1, keepdims=True)

    samples_ref[...] = idx_row
    sel_ref[...] = sel_logit - m_row - jnp.log(s_row)


def kernel(logits, noise):
    b, v = logits.shape
    samples2, sel2 = pl.pallas_call(
        _fused_body,
        grid=(b // _ROWS,),
        in_specs=[
            pl.BlockSpec((_ROWS, v), lambda i: (i, 0)),
            pl.BlockSpec((_ROWS, v), lambda i: (i, 0)),
        ],
        out_specs=[
            pl.BlockSpec((_ROWS, 1), lambda i: (i, 0)),
            pl.BlockSpec((_ROWS, 1), lambda i: (i, 0)),
        ],
        out_shape=[
            jax.ShapeDtypeStruct((b, 1), jnp.int32),
            jax.ShapeDtypeStruct((b, 1), jnp.float32),
        ],
    )(logits, noise)
    return samples2[:, 0], sel2[:, 0]


# speedup vs baseline: 2.3540x; 1.1303x over previous
import jax
import jax.numpy as jnp
from jax import lax
from jax.experimental import pallas as pl

_EPS = 1e-10
_ROWS = 8
_V = 100000
_CW = 512
_UNROLL = 8
_NSETS = 2
_OW = _CW * _UNROLL
_NOUT = _V // _OW
_NEXTRA = (_V - _NOUT * _OW) // _CW
_TAIL = _V - _NOUT * _OW - _NEXTRA * _CW
_BIG = 2**31 - 1


def _fused_body(logits_ref, noise_ref, samples_ref, sel_ref):
    r = _ROWS
    neg_inf = jnp.float32(-jnp.inf)

    def gumbel_perturb(x, n):
        t = jnp.log(n + _EPS)
        w = jnp.log(_EPS - t)
        return x - w

    def update(c, carry):
        bp, bc, bx, m = carry
        col0 = pl.multiple_of(c * _CW, _CW)
        x = logits_ref[:, pl.ds(col0, _CW)]
        n = noise_ref[:, pl.ds(col0, _CW)]
        p = gumbel_perturb(x, n)
        upd = p > bp
        bp = jnp.where(upd, p, bp)
        bc = jnp.where(upd, c, bc)
        bx = jnp.where(upd, x, bx)
        m = jnp.maximum(m, x)
        return bp, bc, bx, m

    def loop_a(i, sets):
        sets = list(sets)
        for j in range(_UNROLL):
            sets[j % _NSETS] = update(i * _UNROLL + j, sets[j % _NSETS])
        return tuple(sets)

    one_set = (
        jnp.full((r, _CW), neg_inf, jnp.float32),
        jnp.zeros((r, _CW), jnp.int32),
        jnp.zeros((r, _CW), jnp.float32),
        jnp.full((r, _CW), neg_inf, jnp.float32),
    )
    sets = lax.fori_loop(0, _NOUT, loop_a, (one_set,) * _NSETS)
    sets = list(sets)
    for j in range(_NEXTRA):
        sets[j % _NSETS] = update(_NOUT * _UNROLL + j, sets[j % _NSETS])

    lane = lax.broadcasted_iota(jnp.int32, (r, _CW), 1)
    bp, bc, bx, m = sets[0]
    bi = bc * _CW + lane
    for k in range(1, _NSETS):
        bp_k, bc_k, bx_k, m_k = sets[k]
        bi_k = bc_k * _CW + lane
        take = (bp_k > bp) | ((bp_k == bp) & (bi_k < bi))
        bp = jnp.where(take, bp_k, bp)
        bi = jnp.where(take, bi_k, bi)
        bx = jnp.where(take, bx_k, bx)
        m = jnp.maximum(m, m_k)

    tail0 = _V - _TAIL
    x_t = logits_ref[:, pl.ds(tail0, _TAIL)]
    n_t = noise_ref[:, pl.ds(tail0, _TAIL)]
    p_t = gumbel_perturb(x_t, n_t)
    iota_t = lax.broadcasted_iota(jnp.int32, (r, _TAIL), 1) + tail0
    pmax_t = jnp.max(p_t, axis=-1, keepdims=True)
    hit_t = p_t == pmax_t
    idx_t = jnp.min(jnp.where(hit_t, iota_t, _BIG), axis=-1, keepdims=True)
    sel_t = jnp.max(jnp.where(iota_t == idx_t, x_t, neg_inf), axis=-1,
                    keepdims=True)
    m_t = jnp.max(x_t, axis=-1, keepdims=True)

    pmax = jnp.max(bp, axis=-1, keepdims=True)
    hit = bp == pmax
    idx_main = jnp.min(jnp.where(hit, bi, _BIG), axis=-1, keepdims=True)
    sel_main = jnp.max(
        jnp.where(hit & (bi == idx_main), bx, neg_inf), axis=-1,
        keepdims=True)

    main_wins = pmax >= pmax_t
    idx_row = jnp.where(main_wins, idx_main, idx_t)
    sel_logit = jnp.where(main_wins, sel_main, sel_t)
    m_row = jnp.maximum(jnp.max(m, axis=-1, keepdims=True), m_t)

    def loop_b(i, s):
        col0 = pl.multiple_of(i * _OW, _OW)
        x = logits_ref[:, pl.ds(col0, _OW)]
        return s + jnp.exp(x - m_row)

    s = lax.fori_loop(0, _NOUT, loop_b,
                      jnp.zeros((r, _OW), jnp.float32))
    s_row = jnp.sum(s, axis=-1, keepdims=True)
    for j in range(_NEXTRA):
        col0 = (_NOUT * _UNROLL + j) * _CW
        x = logits_ref[:, pl.ds(col0, _CW)]
        s_row = s_row + jnp.sum(jnp.exp(x - m_row), axis=-1, keepdims=True)
    s_row = s_row + jnp.sum(jnp.exp(x_t - m_row), axis=-1, keepdims=True)

    samples_ref[...] = idx_row
    sel_ref[...] = sel_logit - m_row - jnp.log(s_row)


def kernel(logits, noise):
    b, v = logits.shape
    samples2, sel2 = pl.pallas_call(
        _fused_body,
        grid=(b // _ROWS,),
        in_specs=[
            pl.BlockSpec((_ROWS, v), lambda i: (i, 0)),
            pl.BlockSpec((_ROWS, v), lambda i: (i, 0)),
        ],
        out_specs=[
            pl.BlockSpec((_ROWS, 1), lambda i: (i, 0)),
            pl.BlockSpec((_ROWS, 1), lambda i: (i, 0)),
        ],
        out_shape=[
            jax.ShapeDtypeStruct((b, 1), jnp.int32),
            jax.ShapeDtypeStruct((b, 1), jnp.float32),
        ],
    )(logits, noise)
    return samples2[:, 0], sel2[:, 0]
